# baseline (speedup 1.0000x reference)
import functools

import jax
import jax.numpy as jnp
from jax import lax
from jax.experimental import pallas as pl
from jax.experimental.pallas import tpu as pltpu
from jax.experimental.pallas import tpu_sc as plsc

_DIM = 64
_N_TOK = 16384
_N_SAMPLES = 8192
_NW = 32
_IDS = _N_TOK + _N_SAMPLES
_B_PER_W = _IDS // _NW
_CHUNK = 128
_KCH = _B_PER_W // _CHUNK


@functools.partial(
    pl.kernel,
    out_type=jax.ShapeDtypeStruct((_IDS, 2 * _DIM), jnp.float32),
    mesh=plsc.VectorSubcoreMesh(core_axis_name="c", subcore_axis_name="s"),
    scratch_types=[
        pltpu.VMEM((_KCH, _CHUNK), jnp.int32),
        pltpu.VMEM((_B_PER_W, 2 * _DIM), jnp.float32),
        pltpu.SemaphoreType.DMA,
    ],
)
def _sc_gather(ids_hbm, table_hbm, out_hbm, idx_v, rows_v, sem):
    wid = lax.axis_index("s") * 2 + lax.axis_index("c")
    pltpu.sync_copy(ids_hbm.at[wid], idx_v)
    cps = [
        pltpu.async_copy(
            table_hbm.at[idx_v.at[k]],
            rows_v.at[pl.ds(k * _CHUNK, _CHUNK)],
            sem,
        )
        for k in range(_KCH)
    ]
    for cp in cps:
        cp.wait()
    pltpu.sync_copy(rows_v, out_hbm.at[pl.ds(wid * _B_PER_W, _B_PER_W)])


_TILE_N = 256
_GRID = _N_TOK // _TILE_N


def _tc_loss_body(x_ref, pe_ref, t_ref, nid_ref, ne_ref, acc_ref):
    i = pl.program_id(0)
    x = x_ref[...]
    z = lax.dot_general(
        x, ne_ref[...], (((1,), (1,)), ((), ())),
        preferred_element_type=jnp.float32,
    )
    t = t_ref[...]
    hits = nid_ref[0] == t
    z = jnp.where(hits, jnp.float32(-1e9), z)
    pos = jnp.sum(x * pe_ref[...], axis=1, keepdims=True)
    s = jnp.sum(jnp.exp(z), axis=1, keepdims=True) + jnp.exp(pos)
    loss = jnp.log(s) - pos
    mask = t != -100
    loss = jnp.where(mask, loss, jnp.float32(0.0))
    part = jnp.sum(loss, axis=(0, 1), keepdims=True)
    cnt = jnp.sum(mask.astype(jnp.float32), axis=(0, 1), keepdims=True)
    vec = jnp.concatenate([part, cnt], axis=1)

    @pl.when(i == 0)
    def _init():
        acc_ref[...] = jnp.zeros_like(acc_ref)

    acc_ref[...] += vec


_tc_loss = pl.pallas_call(
    _tc_loss_body,
    grid=(_GRID,),
    in_specs=[
        pl.BlockSpec((_TILE_N, _DIM), lambda i: (i, 0)),
        pl.BlockSpec((_TILE_N, _DIM), lambda i: (i, 0)),
        pl.BlockSpec((_TILE_N, 1), lambda i: (i, 0)),
        pl.BlockSpec((1, 1, _N_SAMPLES), lambda i: (0, 0, 0)),
        pl.BlockSpec((_N_SAMPLES, _DIM), lambda i: (0, 0)),
    ],
    out_specs=pl.BlockSpec((1, 2), lambda i: (0, 0)),
    out_shape=jax.ShapeDtypeStruct((1, 2), jnp.float32),
)


def kernel(inputs, target_ids, table, neg_ids):
    t = target_ids[:, 0]
    ids = jnp.concatenate([t, neg_ids])
    phys = (ids >> 1).reshape(_NW, _KCH, _CHUNK)
    table128 = table.reshape(-1, 2 * _DIM)
    rows128 = _sc_gather(phys, table128)
    odd = (ids & 1)[:, None] == 1
    rows = jnp.where(odd, rows128[:, _DIM:], rows128[:, :_DIM])
    pos_emb = rows[:_N_TOK]
    neg_emb = rows[_N_TOK:]
    acc = _tc_loss(
        inputs, pos_emb, target_ids,
        neg_ids.reshape(1, 1, _N_SAMPLES), neg_emb,
    )
    loss = acc[0, 0] / acc[0, 1]
    return (jnp.asarray(0), loss)


# speedup vs baseline: 1.4442x; 1.4442x over previous
import functools

import jax
import jax.numpy as jnp
from jax import lax
from jax.experimental import pallas as pl
from jax.experimental.pallas import tpu as pltpu
from jax.experimental.pallas import tpu_sc as plsc

_DIM = 64
_N_TOK = 16384
_N_SAMPLES = 8192
_NW = 32
_IDS = _N_TOK + _N_SAMPLES
_B_PER_W = _IDS // _NW
_CHUNK = 128
_KCH = _B_PER_W // _CHUNK


_ROWS_PER_STEP = 16
_STEPS = _B_PER_W // _ROWS_PER_STEP


@functools.partial(
    pl.kernel,
    out_type=jax.ShapeDtypeStruct((_IDS, _DIM), jnp.float32),
    mesh=plsc.VectorSubcoreMesh(core_axis_name="c", subcore_axis_name="s"),
    scratch_types=[
        pltpu.VMEM((_B_PER_W,), jnp.int32),
        pltpu.VMEM((_B_PER_W, _DIM), jnp.float32),
        pltpu.SemaphoreType.DMA,
    ],
)
def _sc_gather(ids_hbm, table_hbm, out_hbm, idx_v, rows_v, sem):
    wid = lax.axis_index("s") * 2 + lax.axis_index("c")
    pltpu.sync_copy(ids_hbm.at[wid], idx_v)

    def step(g, _):
        base = g * _ROWS_PER_STEP
        vec = idx_v[pl.ds(base, _ROWS_PER_STEP)]
        cps = []
        for j in range(_ROWS_PER_STEP):
            r = vec[j]
            cps.append(pltpu.async_copy(table_hbm.at[r], rows_v.at[base + j], sem))
        for cp in cps:
            cp.wait()
        return 0

    lax.fori_loop(0, _STEPS, step, 0)
    pltpu.sync_copy(rows_v, out_hbm.at[pl.ds(wid * _B_PER_W, _B_PER_W)])


_TILE_N = 256
_GRID = _N_TOK // _TILE_N


def _tc_loss_body(x_ref, pe_ref, t_ref, nid_ref, ne_ref, acc_ref):
    i = pl.program_id(0)
    x = x_ref[...]
    z = lax.dot_general(
        x, ne_ref[...], (((1,), (1,)), ((), ())),
        preferred_element_type=jnp.float32,
    )
    t = t_ref[...]
    hits = nid_ref[0] == t
    z = jnp.where(hits, jnp.float32(-1e9), z)
    pos = jnp.sum(x * pe_ref[...], axis=1, keepdims=True)
    s = jnp.sum(jnp.exp(z), axis=1, keepdims=True) + jnp.exp(pos)
    loss = jnp.log(s) - pos
    mask = t != -100
    loss = jnp.where(mask, loss, jnp.float32(0.0))
    part = jnp.sum(loss, axis=(0, 1), keepdims=True)
    cnt = jnp.sum(mask.astype(jnp.float32), axis=(0, 1), keepdims=True)
    vec = jnp.concatenate([part, cnt], axis=1)

    @pl.when(i == 0)
    def _init():
        acc_ref[...] = jnp.zeros_like(acc_ref)

    acc_ref[...] += vec


_tc_loss = pl.pallas_call(
    _tc_loss_body,
    grid=(_GRID,),
    in_specs=[
        pl.BlockSpec((_TILE_N, _DIM), lambda i: (i, 0)),
        pl.BlockSpec((_TILE_N, _DIM), lambda i: (i, 0)),
        pl.BlockSpec((_TILE_N, 1), lambda i: (i, 0)),
        pl.BlockSpec((1, 1, _N_SAMPLES), lambda i: (0, 0, 0)),
        pl.BlockSpec((_N_SAMPLES, _DIM), lambda i: (0, 0)),
    ],
    out_specs=pl.BlockSpec((1, 2), lambda i: (0, 0)),
    out_shape=jax.ShapeDtypeStruct((1, 2), jnp.float32),
)


def kernel(inputs, target_ids, table, neg_ids):
    t = target_ids[:, 0]
    ids = jnp.concatenate([t, neg_ids]).reshape(_NW, _B_PER_W)
    rows = _sc_gather(ids, table)
    pos_emb = rows[:_N_TOK]
    neg_emb = rows[_N_TOK:]
    acc = _tc_loss(
        inputs, pos_emb, target_ids,
        neg_ids.reshape(1, 1, _N_SAMPLES), neg_emb,
    )
    loss = acc[0, 0] / acc[0, 1]
    return (jnp.asarray(0), loss)
